# baseline (speedup 1.0000x reference)
import functools
import math

import jax
import jax.numpy as jnp
from jax import lax
from jax.experimental import pallas as pl
from jax.experimental.pallas import tpu as pltpu
from jax.experimental.pallas import tpu_sc as plsc

BATCH = 4096
SEQ = 200
D = 64
DP = 128
NW = 32
CHUNK = 128
NIN = 5
AHEAD = 3

_mesh = plsc.VectorSubcoreMesh(core_axis_name="c", subcore_axis_name="s")


def _make_half(seq_len):
    rows = BATCH * seq_len
    rpw = rows // NW
    cpw = rpw // CHUNK
    blk = math.lcm(CHUNK, seq_len) // CHUNK
    assert cpw % blk == 0 and blk % NIN == 0

    @functools.partial(
        pl.kernel,
        out_type=jax.ShapeDtypeStruct((rows, DP), jnp.float32),
        mesh=_mesh,
        scratch_types=[
            pltpu.VMEM((rpw,), jnp.int32),
            pltpu.VMEM((seq_len // 2, DP), jnp.float32),
            [pltpu.VMEM((CHUNK, DP), jnp.float32) for _ in range(NIN)],
            [pltpu.SemaphoreType.DMA for _ in range(NIN)],
            [pltpu.SemaphoreType.DMA for _ in range(NIN)],
        ],
    )
    def _half(seq_hbm, tok_hbm, pos_hbm, out_hbm,
              idx_v, pos_v, in_bufs, sem_in, sem_out):
        wid = lax.axis_index("s") * 2 + lax.axis_index("c")
        flat_base = wid * rpw

        pltpu.sync_copy(seq_hbm.at[pl.ds(flat_base, rpw)], idx_v)
        pltpu.sync_copy(pos_hbm, pos_v)

        def gather_desc(g, bi):
            idx = idx_v.at[pl.ds(g * CHUNK, CHUNK)]
            return pltpu.make_async_copy(
                tok_hbm.at[idx], in_bufs[bi], sem_in[bi])

        def out_desc(g, bi):
            return pltpu.make_async_copy(
                in_bufs[bi], out_hbm.at[pl.ds(flat_base + g * CHUNK, CHUNK)],
                sem_out[bi])

        def add_seg(bi, row0, nrows, phase):
            @pl.loop(0, nrows // 4)
            def _i4(i4):
                for r in range(4):
                    prow = phase // 2 + i4 * 2 + r // 2
                    pcol = (r % 2) * D
                    for j in range(D // 16):
                        pvec = pos_v[prow, pl.ds(pcol + j * 16, 16)]
                        plsc.addupdate(
                            in_bufs[bi].at[row0 + i4 * 4 + r, pl.ds(j * 16, 16)],
                            pvec)

        for g in range(AHEAD):
            gather_desc(g, g % NIN).start()

        @pl.loop(0, cpw // blk)
        def _blk(b):
            g0 = b * blk
            for k in range(blk):
                g = g0 + k
                bi = k % NIN
                b3 = (k + AHEAD) % NIN
                phase = (k * CHUNK) % seq_len
                gather_desc(g, bi).wait()

                @pl.when(g + AHEAD < cpw)
                def _(g=g, b3=b3):
                    @pl.when(g >= NIN - AHEAD)
                    def _():
                        out_desc(g - (NIN - AHEAD), b3).wait()

                    gather_desc(g + AHEAD, b3).start()

                left, row0, p = CHUNK, 0, phase
                while left:
                    seg = min(left, seq_len - p)
                    add_seg(bi, row0, seg, p)
                    row0, left, p = row0 + seg, left - seg, 0
                out_desc(g, bi).start()

        for g in range(cpw - NIN, cpw):
            out_desc(g, g % NIN).wait()

    return _half


_HALF = _make_half(SEQ // 2)


def kernel(seq, token_table, pos_table):
    tok_p = jnp.pad(token_table, ((0, 0), (0, DP - D)))
    hs = SEQ // 2
    halves = []
    for h in range(2):
        seq_h = seq[:, h * hs:(h + 1) * hs].reshape(BATCH * hs)
        pos_h = pos_table[h * hs:(h + 1) * hs]
        pos_p = jnp.concatenate([pos_h[0::2], pos_h[1::2]], axis=1)
        out_h = _HALF(seq_h, tok_p, pos_p)
        halves.append(out_h[:, :D].reshape(BATCH, hs, D))
    return jnp.concatenate(halves, axis=1)
